# baseline (speedup 1.0000x reference)
import jax
import jax.numpy as jnp
from jax.experimental import pallas as pl
from jax.experimental.pallas import tpu as pltpu

_BM = 512
_NBUF = 8


def _x_copy(x_hbm, xbuf, xsem, k, slot):
    return pltpu.make_async_copy(
        x_hbm.at[pl.ds(k * _BM, _BM), :], xbuf.at[slot], xsem.at[slot])


def _o_copy(o_hbm, obuf, osem, k, slot):
    return pltpu.make_async_copy(
        obuf.at[slot], o_hbm.at[pl.ds(k * _BM, _BM), :], osem.at[slot])


def _affine_kernel(x_hbm, w_hbm, b_ref, o_hbm,
                   xbuf, obuf, wtmp, wb, xsem, osem, wsem):
    n_steps = x_hbm.shape[0] // _BM

    w_dma = pltpu.make_async_copy(w_hbm, wtmp, wsem)
    w_dma.start()
    for k in range(_NBUF):
        _x_copy(x_hbm, xbuf, xsem, k, k).start()
    w_dma.wait()
    wb[...] = wtmp[...].astype(jnp.bfloat16)
    bias = b_ref[...]

    def step(k, carry):
        slot = jax.lax.rem(k, _NBUF)
        _x_copy(x_hbm, xbuf, xsem, k, slot).wait()

        @pl.when(k >= _NBUF)
        def _drain_old_write():
            _o_copy(o_hbm, obuf, osem, k - _NBUF, slot).wait()

        acc = jax.lax.dot_general(
            xbuf[slot].astype(jnp.bfloat16), wb[...],
            dimension_numbers=(((1,), (1,)), ((), ())),
            preferred_element_type=jnp.float32,
        )
        obuf[slot] = acc + bias
        _o_copy(o_hbm, obuf, osem, k, slot).start()

        @pl.when(k + _NBUF < n_steps)
        def _prefetch_next():
            _x_copy(x_hbm, xbuf, xsem, k + _NBUF, slot).start()

        return carry

    jax.lax.fori_loop(0, n_steps, step, 0)

    for s in range(_NBUF):
        k = n_steps - _NBUF + s
        _o_copy(o_hbm, obuf, osem, k, jax.lax.rem(k, _NBUF)).wait()


def kernel(flat, cu_seqlens, W, b):
    del cu_seqlens
    M, d = flat.shape
    return pl.pallas_call(
        _affine_kernel,
        in_specs=[
            pl.BlockSpec(memory_space=pltpu.MemorySpace.HBM),
            pl.BlockSpec(memory_space=pltpu.MemorySpace.HBM),
            pl.BlockSpec(memory_space=pltpu.MemorySpace.VMEM),
        ],
        out_specs=pl.BlockSpec(memory_space=pltpu.MemorySpace.HBM),
        out_shape=jax.ShapeDtypeStruct((M, d), jnp.float32),
        scratch_shapes=[
            pltpu.VMEM((_NBUF, _BM, d), jnp.float32),
            pltpu.VMEM((_NBUF, _BM, d), jnp.float32),
            pltpu.VMEM((d, d), jnp.float32),
            pltpu.VMEM((d, d), jnp.bfloat16),
            pltpu.SemaphoreType.DMA((_NBUF,)),
            pltpu.SemaphoreType.DMA((_NBUF,)),
            pltpu.SemaphoreType.DMA,
        ],
    )(flat, W, b.reshape(1, d))


# speedup vs baseline: 1.1047x; 1.1047x over previous
import jax
import jax.numpy as jnp
from jax.experimental import pallas as pl
from jax.experimental.pallas import tpu as pltpu

_BM = 1024
_NBUF = 6


def _x_copy(x_hbm, xbuf, xsem, k, slot):
    return pltpu.make_async_copy(
        x_hbm.at[pl.ds(k * _BM, _BM), :], xbuf.at[slot], xsem.at[slot])


def _o_copy(o_hbm, obuf, osem, k, slot):
    return pltpu.make_async_copy(
        obuf.at[slot], o_hbm.at[pl.ds(k * _BM, _BM), :], osem.at[slot])


def _affine_kernel(x_hbm, w_hbm, b_ref, o_hbm,
                   xbuf, obuf, wtmp, wb, xsem, osem, wsem):
    n_steps = x_hbm.shape[0] // _BM

    w_dma = pltpu.make_async_copy(w_hbm, wtmp, wsem)
    w_dma.start()
    for k in range(_NBUF):
        _x_copy(x_hbm, xbuf, xsem, k, k).start()
    w_dma.wait()
    wb[...] = wtmp[...].astype(jnp.bfloat16)
    bias = b_ref[...]

    def step(k, carry):
        slot = jax.lax.rem(k, _NBUF)
        _x_copy(x_hbm, xbuf, xsem, k, slot).wait()

        @pl.when(k >= _NBUF)
        def _drain_old_write():
            _o_copy(o_hbm, obuf, osem, k - _NBUF, slot).wait()

        acc = jax.lax.dot_general(
            xbuf[slot].astype(jnp.bfloat16), wb[...],
            dimension_numbers=(((1,), (1,)), ((), ())),
            preferred_element_type=jnp.float32,
        )
        obuf[slot] = acc + bias
        _o_copy(o_hbm, obuf, osem, k, slot).start()

        @pl.when(k + _NBUF < n_steps)
        def _prefetch_next():
            _x_copy(x_hbm, xbuf, xsem, k + _NBUF, slot).start()

        return carry

    jax.lax.fori_loop(0, n_steps, step, 0)

    for s in range(_NBUF):
        k = n_steps - _NBUF + s
        _o_copy(o_hbm, obuf, osem, k, jax.lax.rem(k, _NBUF)).wait()


def kernel(flat, cu_seqlens, W, b):
    del cu_seqlens
    M, d = flat.shape
    return pl.pallas_call(
        _affine_kernel,
        in_specs=[
            pl.BlockSpec(memory_space=pltpu.MemorySpace.HBM),
            pl.BlockSpec(memory_space=pltpu.MemorySpace.HBM),
            pl.BlockSpec(memory_space=pltpu.MemorySpace.VMEM),
        ],
        out_specs=pl.BlockSpec(memory_space=pltpu.MemorySpace.HBM),
        out_shape=jax.ShapeDtypeStruct((M, d), jnp.float32),
        scratch_shapes=[
            pltpu.VMEM((_NBUF, _BM, d), jnp.float32),
            pltpu.VMEM((_NBUF, _BM, d), jnp.float32),
            pltpu.VMEM((d, d), jnp.float32),
            pltpu.VMEM((d, d), jnp.bfloat16),
            pltpu.SemaphoreType.DMA((_NBUF,)),
            pltpu.SemaphoreType.DMA((_NBUF,)),
            pltpu.SemaphoreType.DMA,
        ],
    )(flat, W, b.reshape(1, d))


# speedup vs baseline: 1.1513x; 1.0421x over previous
import jax
import jax.numpy as jnp
from jax.experimental import pallas as pl
from jax.experimental.pallas import tpu as pltpu

_BM = 2048
_NBUF = 3


def _x_copy(x_hbm, xbuf, xsem, k, slot):
    return pltpu.make_async_copy(
        x_hbm.at[pl.ds(k * _BM, _BM), :], xbuf.at[slot], xsem.at[slot])


def _o_copy(o_hbm, obuf, osem, k, slot):
    return pltpu.make_async_copy(
        obuf.at[slot], o_hbm.at[pl.ds(k * _BM, _BM), :], osem.at[slot])


def _affine_kernel(x_hbm, w_hbm, b_ref, o_hbm,
                   xbuf, obuf, wb, xsem, osem, wsem):
    n_steps = x_hbm.shape[0] // _BM

    d = w_hbm.shape[0]
    w_land = obuf.at[0, pl.ds(0, d), :]
    w_dma = pltpu.make_async_copy(w_hbm, w_land, wsem)
    w_dma.start()
    for k in range(_NBUF):
        _x_copy(x_hbm, xbuf, xsem, k, k).start()
    w_dma.wait()
    wb[...] = obuf[0, pl.ds(0, d), :].astype(jnp.bfloat16)
    bias = b_ref[...]

    def step(k, carry):
        slot = jax.lax.rem(k, _NBUF)
        _x_copy(x_hbm, xbuf, xsem, k, slot).wait()

        @pl.when(k >= _NBUF)
        def _drain_old_write():
            _o_copy(o_hbm, obuf, osem, k - _NBUF, slot).wait()

        acc = jax.lax.dot_general(
            xbuf[slot].astype(jnp.bfloat16), wb[...],
            dimension_numbers=(((1,), (1,)), ((), ())),
            preferred_element_type=jnp.float32,
        )
        obuf[slot] = acc + bias
        _o_copy(o_hbm, obuf, osem, k, slot).start()

        @pl.when(k + _NBUF < n_steps)
        def _prefetch_next():
            _x_copy(x_hbm, xbuf, xsem, k + _NBUF, slot).start()

        return carry

    jax.lax.fori_loop(0, n_steps, step, 0)

    for s in range(_NBUF):
        k = n_steps - _NBUF + s
        _o_copy(o_hbm, obuf, osem, k, jax.lax.rem(k, _NBUF)).wait()


def kernel(flat, cu_seqlens, W, b):
    del cu_seqlens
    M, d = flat.shape
    return pl.pallas_call(
        _affine_kernel,
        in_specs=[
            pl.BlockSpec(memory_space=pltpu.MemorySpace.HBM),
            pl.BlockSpec(memory_space=pltpu.MemorySpace.HBM),
            pl.BlockSpec(memory_space=pltpu.MemorySpace.VMEM),
        ],
        out_specs=pl.BlockSpec(memory_space=pltpu.MemorySpace.HBM),
        out_shape=jax.ShapeDtypeStruct((M, d), jnp.float32),
        scratch_shapes=[
            pltpu.VMEM((_NBUF, _BM, d), jnp.float32),
            pltpu.VMEM((_NBUF, _BM, d), jnp.float32),
            pltpu.VMEM((d, d), jnp.bfloat16),
            pltpu.SemaphoreType.DMA((_NBUF,)),
            pltpu.SemaphoreType.DMA((_NBUF,)),
            pltpu.SemaphoreType.DMA,
        ],
    )(flat, W, b.reshape(1, d))
